# baseline (speedup 1.0000x reference)
import functools

import jax
import jax.numpy as jnp
from jax import lax
from jax.experimental import pallas as pl
from jax.experimental.pallas import tpu as pltpu
from jax.experimental.pallas import tpu_sc as plsc

_N = 10000
_NP = 10240
_D = 128
_E = 320000
_NC = 2
_NS = 16
_NW = _NC * _NS
_CH = 128
_K = 80
_RCH = _E // _CH
_RPT = _NP // _NS


_NB = 2


def _sc_agg_body(*refs, with_counts):
    if with_counts:
        (y_hbm, src_hbm, dst_hbm, zr_hbm, zc_hbm, agg_hbm, cnt_hbm,
         acc_sh, cnt_sh, rows_v, sidx, didx, ones_v, gsem, ssem, csem, isem) = refs
    else:
        (y_hbm, src_hbm, dst_hbm, zr_hbm, agg_hbm,
         acc_sh, rows_v, sidx, didx, gsem, ssem, isem) = refs

    c = lax.axis_index("c")
    s = lax.axis_index("s")
    wid = s * _NC + c
    base = s * _RPT
    ebase = wid * _K * _CH
    nchunks = jnp.minimum(_K, jnp.maximum(_RCH - wid * _K, 0))

    pltpu.sync_copy(zr_hbm, rows_v[0])
    for i in range(_RPT // _CH):
        pltpu.sync_copy(rows_v[0], acc_sh.at[pl.ds(base + i * _CH, _CH)])
    if with_counts:
        pltpu.sync_copy(zc_hbm, cnt_sh.at[pl.ds(base, _RPT)])
        for i in range(_CH // 16):
            ones_v[pl.ds(i * 16, 16)] = jnp.full((16,), 1.0, jnp.float32)
    plsc.subcore_barrier()

    def idx_start(j, b, p):
        off = ebase + j * _CH
        pltpu.async_copy(src_hbm.at[pl.ds(off, _CH)], sidx[p][b], isem[p][b])
        pltpu.async_copy(dst_hbm.at[pl.ds(off, _CH)], didx[p][b], isem[p][b])

    def idx_wait(b, p):
        pltpu.make_async_copy(src_hbm.at[pl.ds(0, _CH)], sidx[p][b], isem[p][b]).wait()
        pltpu.make_async_copy(dst_hbm.at[pl.ds(0, _CH)], didx[p][b], isem[p][b]).wait()

    def gather_start(b, p):
        pltpu.async_copy(y_hbm.at[sidx[p][b]], rows_v[b], gsem[b])

    def gather_wait(b):
        pltpu.make_async_copy(y_hbm.at[pl.ds(0, _CH)], rows_v[b], gsem[b]).wait()

    def scatter_start(b, p):
        pltpu.async_copy(rows_v[b], acc_sh.at[didx[p][b]], ssem[b], add=True)
        if with_counts:
            pltpu.async_copy(ones_v, cnt_sh.at[didx[p][b]], csem[b], add=True)

    def scatter_wait(b):
        pltpu.make_async_copy(rows_v[b], acc_sh.at[pl.ds(0, _CH)], ssem[b]).wait()
        if with_counts:
            pltpu.make_async_copy(ones_v, cnt_sh.at[pl.ds(0, _CH)], csem[b]).wait()

    def halfgroup(base_chunk, p):
        for b in range(_NB):
            j = base_chunk + b
            gather_wait(b)
            scatter_start(b, p)

            @pl.when(j + _NB < nchunks)
            def _():
                idx_start(j + _NB, b, 1 - p)
        for b in range(_NB):
            j = base_chunk + b
            scatter_wait(b)

            @pl.when(j + _NB < nchunks)
            def _():
                idx_wait(b, 1 - p)
                gather_start(b, 1 - p)

    def group(jj, carry):
        q0 = jj * 2 * _NB
        halfgroup(q0, 0)
        halfgroup(q0 + _NB, 1)
        return carry

    for b in range(_NB):
        idx_start(b, b, 0)
    for b in range(_NB):
        idx_wait(b, 0)
        gather_start(b, 0)
    lax.fori_loop(0, nchunks // (2 * _NB), group, 0)
    plsc.subcore_barrier()

    off = c * _NP + base
    pltpu.sync_copy(acc_sh.at[pl.ds(base, _RPT)], agg_hbm.at[pl.ds(off, _RPT)])
    if with_counts:
        pltpu.sync_copy(cnt_sh.at[pl.ds(base, _RPT)], cnt_hbm.at[pl.ds(off, _RPT)])


_SC_MESH = plsc.VectorSubcoreMesh(core_axis_name="c", subcore_axis_name="s")


def _sc_agg_with_counts(y, src, dst, zr, zc):
    return pl.kernel(
        functools.partial(_sc_agg_body, with_counts=True),
        out_type=(jax.ShapeDtypeStruct((_NC * _NP, _D), jnp.float32),
                  jax.ShapeDtypeStruct((_NC * _NP,), jnp.float32)),
        mesh=_SC_MESH,
        scratch_types=[
            pltpu.VMEM_SHARED((_NP, _D), jnp.float32),
            pltpu.VMEM_SHARED((_NP,), jnp.float32),
            [pltpu.VMEM((_CH, _D), jnp.float32) for _ in range(_NB)],
            [[pltpu.VMEM((_CH,), jnp.int32) for _ in range(_NB)] for _ in range(2)],
            [[pltpu.VMEM((_CH,), jnp.int32) for _ in range(_NB)] for _ in range(2)],
            pltpu.VMEM((_CH,), jnp.float32),
            [pltpu.SemaphoreType.DMA for _ in range(_NB)],
            [pltpu.SemaphoreType.DMA for _ in range(_NB)],
            [pltpu.SemaphoreType.DMA for _ in range(_NB)],
            [[pltpu.SemaphoreType.DMA for _ in range(_NB)] for _ in range(2)],
        ],
    )(y, src, dst, zr, zc)


def _sc_agg(y, src, dst, zr):
    return pl.kernel(
        functools.partial(_sc_agg_body, with_counts=False),
        out_type=jax.ShapeDtypeStruct((_NC * _NP, _D), jnp.float32),
        mesh=_SC_MESH,
        scratch_types=[
            pltpu.VMEM_SHARED((_NP, _D), jnp.float32),
            [pltpu.VMEM((_CH, _D), jnp.float32) for _ in range(_NB)],
            [[pltpu.VMEM((_CH,), jnp.int32) for _ in range(_NB)] for _ in range(2)],
            [[pltpu.VMEM((_CH,), jnp.int32) for _ in range(_NB)] for _ in range(2)],
            [pltpu.SemaphoreType.DMA for _ in range(_NB)],
            [pltpu.SemaphoreType.DMA for _ in range(_NB)],
            [[pltpu.SemaphoreType.DMA for _ in range(_NB)] for _ in range(2)],
        ],
    )(y, src, dst, zr)


_BLK = 1024


def _tc_combine_body(agg_ref, cnt_ref, x_ref, wl_ref, wr_ref, b_ref, o_ref, *, relu):
    a = agg_ref[0] + agg_ref[1]
    cntv = cnt_ref[0] + cnt_ref[1]
    inv = 1.0 / jnp.maximum(cntv, 1.0)
    mean = a * inv[:, None]
    r = (jnp.dot(mean, wl_ref[...], preferred_element_type=jnp.float32)
         + jnp.dot(x_ref[...], wr_ref[...], preferred_element_type=jnp.float32)
         + b_ref[...])
    o_ref[...] = jnp.maximum(r, 0.0) if relu else r


def _tc_combine(agg2, cnt2, xin, wlT, wrT, b, relu):
    return pl.pallas_call(
        functools.partial(_tc_combine_body, relu=relu),
        grid=(pl.cdiv(_N, _BLK),),
        in_specs=[
            pl.BlockSpec((_NC, _BLK, _D), lambda i: (0, i, 0)),
            pl.BlockSpec((_NC, _BLK), lambda i: (0, i)),
            pl.BlockSpec((_BLK, _D), lambda i: (i, 0)),
            pl.BlockSpec((_D, _D), lambda i: (0, 0)),
            pl.BlockSpec((_D, _D), lambda i: (0, 0)),
            pl.BlockSpec((1, _D), lambda i: (0, 0)),
        ],
        out_specs=pl.BlockSpec((_BLK, _D), lambda i: (i, 0)),
        out_shape=jax.ShapeDtypeStruct((_N, _D), jnp.float32),
    )(agg2, cnt2, xin, wlT, wrT, b)


def kernel(x, edge_index, W1_l, W1_r, b1, W2_l, W2_r, b2):
    zr = jnp.zeros((_CH, _D), jnp.float32)
    zc = jnp.zeros((_RPT,), jnp.float32)

    src = edge_index[0]
    dst = edge_index[1]
    agg1, cnt = _sc_agg_with_counts(x, src, dst, zr, zc)
    h = _tc_combine(agg1.reshape(_NC, _NP, _D), cnt.reshape(_NC, _NP),
                    x, W1_l.T, W1_r.T, b1.reshape(1, _D), relu=True)
    agg2 = _sc_agg(h, src, dst, zr)
    out = _tc_combine(agg2.reshape(_NC, _NP, _D), cnt.reshape(_NC, _NP),
                      h, W2_l.T, W2_r.T, b2.reshape(1, _D), relu=False)
    return out


# speedup vs baseline: 1.1794x; 1.1794x over previous
import functools

import jax
import jax.numpy as jnp
from jax import lax
from jax.experimental import pallas as pl
from jax.experimental.pallas import tpu as pltpu
from jax.experimental.pallas import tpu_sc as plsc

_N = 10000
_NP = 10240
_D = 128
_E = 320000
_NC = 2
_NS = 16
_NW = _NC * _NS
_CH = 64
_K = 160
_RCH = _E // _CH
_RPT = _NP // _NS


_NB = 5


def _sc_agg_body(*refs, with_counts):
    if with_counts:
        (y_hbm, src_hbm, dst_hbm, zr_hbm, zc_hbm, agg_hbm, cnt_hbm,
         acc_sh, cnt_sh, rows_v, sidx, didx, ones_v, gsem, ssem, csem, isem) = refs
    else:
        (y_hbm, src_hbm, dst_hbm, zr_hbm, agg_hbm,
         acc_sh, rows_v, sidx, didx, gsem, ssem, isem) = refs

    c = lax.axis_index("c")
    s = lax.axis_index("s")
    wid = s * _NC + c
    base = s * _RPT
    ebase = wid * _K * _CH
    nchunks = jnp.minimum(_K, jnp.maximum(_RCH - wid * _K, 0))

    pltpu.sync_copy(zr_hbm, rows_v[0])
    for i in range(_RPT // _CH):
        pltpu.sync_copy(rows_v[0], acc_sh.at[pl.ds(base + i * _CH, _CH)])
    if with_counts:
        pltpu.sync_copy(zc_hbm, cnt_sh.at[pl.ds(base, _RPT)])
        for i in range(_CH // 16):
            ones_v[pl.ds(i * 16, 16)] = jnp.full((16,), 1.0, jnp.float32)
    plsc.subcore_barrier()

    def idx_start(j, b, p):
        off = ebase + j * _CH
        pltpu.async_copy(src_hbm.at[pl.ds(off, _CH)], sidx[p][b], isem[p][b])
        pltpu.async_copy(dst_hbm.at[pl.ds(off, _CH)], didx[p][b], isem[p][b])

    def idx_wait(b, p):
        pltpu.make_async_copy(src_hbm.at[pl.ds(0, _CH)], sidx[p][b], isem[p][b]).wait()
        pltpu.make_async_copy(dst_hbm.at[pl.ds(0, _CH)], didx[p][b], isem[p][b]).wait()

    def gather_start(b, p):
        pltpu.async_copy(y_hbm.at[sidx[p][b]], rows_v[b], gsem[b])

    def gather_wait(b):
        pltpu.make_async_copy(y_hbm.at[pl.ds(0, _CH)], rows_v[b], gsem[b]).wait()

    def scatter_start(b, p):
        pltpu.async_copy(rows_v[b], acc_sh.at[didx[p][b]], ssem[b], add=True)
        if with_counts:
            pltpu.async_copy(ones_v, cnt_sh.at[didx[p][b]], csem[b], add=True)

    def scatter_wait(b):
        pltpu.make_async_copy(rows_v[b], acc_sh.at[pl.ds(0, _CH)], ssem[b]).wait()
        if with_counts:
            pltpu.make_async_copy(ones_v, cnt_sh.at[pl.ds(0, _CH)], csem[b]).wait()

    def halfgroup(base_chunk, p):
        for b in range(_NB):
            j = base_chunk + b
            gather_wait(b)
            scatter_start(b, p)

            @pl.when(j + _NB < nchunks)
            def _():
                idx_start(j + _NB, b, 1 - p)
        for b in range(_NB):
            j = base_chunk + b
            scatter_wait(b)

            @pl.when(j + _NB < nchunks)
            def _():
                idx_wait(b, 1 - p)
                gather_start(b, 1 - p)

    def group(jj, carry):
        q0 = jj * 2 * _NB
        halfgroup(q0, 0)
        halfgroup(q0 + _NB, 1)
        return carry

    for b in range(_NB):
        idx_start(b, b, 0)
    for b in range(_NB):
        idx_wait(b, 0)
        gather_start(b, 0)
    lax.fori_loop(0, nchunks // (2 * _NB), group, 0)
    plsc.subcore_barrier()

    off = c * _NP + base
    pltpu.sync_copy(acc_sh.at[pl.ds(base, _RPT)], agg_hbm.at[pl.ds(off, _RPT)])
    if with_counts:
        pltpu.sync_copy(cnt_sh.at[pl.ds(base, _RPT)], cnt_hbm.at[pl.ds(off, _RPT)])


_SC_MESH = plsc.VectorSubcoreMesh(core_axis_name="c", subcore_axis_name="s")


def _sc_agg_with_counts(y, src, dst, zr, zc):
    return pl.kernel(
        functools.partial(_sc_agg_body, with_counts=True),
        out_type=(jax.ShapeDtypeStruct((_NC * _NP, _D), jnp.float32),
                  jax.ShapeDtypeStruct((_NC * _NP,), jnp.float32)),
        mesh=_SC_MESH,
        scratch_types=[
            pltpu.VMEM_SHARED((_NP, _D), jnp.float32),
            pltpu.VMEM_SHARED((_NP,), jnp.float32),
            [pltpu.VMEM((_CH, _D), jnp.float32) for _ in range(_NB)],
            [[pltpu.VMEM((_CH,), jnp.int32) for _ in range(_NB)] for _ in range(2)],
            [[pltpu.VMEM((_CH,), jnp.int32) for _ in range(_NB)] for _ in range(2)],
            pltpu.VMEM((_CH,), jnp.float32),
            [pltpu.SemaphoreType.DMA for _ in range(_NB)],
            [pltpu.SemaphoreType.DMA for _ in range(_NB)],
            [pltpu.SemaphoreType.DMA for _ in range(_NB)],
            [[pltpu.SemaphoreType.DMA for _ in range(_NB)] for _ in range(2)],
        ],
    )(y, src, dst, zr, zc)


def _sc_agg(y, src, dst, zr):
    return pl.kernel(
        functools.partial(_sc_agg_body, with_counts=False),
        out_type=jax.ShapeDtypeStruct((_NC * _NP, _D), jnp.float32),
        mesh=_SC_MESH,
        scratch_types=[
            pltpu.VMEM_SHARED((_NP, _D), jnp.float32),
            [pltpu.VMEM((_CH, _D), jnp.float32) for _ in range(_NB)],
            [[pltpu.VMEM((_CH,), jnp.int32) for _ in range(_NB)] for _ in range(2)],
            [[pltpu.VMEM((_CH,), jnp.int32) for _ in range(_NB)] for _ in range(2)],
            [pltpu.SemaphoreType.DMA for _ in range(_NB)],
            [pltpu.SemaphoreType.DMA for _ in range(_NB)],
            [[pltpu.SemaphoreType.DMA for _ in range(_NB)] for _ in range(2)],
        ],
    )(y, src, dst, zr)


_BLK = 1024


def _tc_combine_body(agg_ref, cnt_ref, x_ref, wl_ref, wr_ref, b_ref, o_ref, *, relu):
    a = agg_ref[0] + agg_ref[1]
    cntv = cnt_ref[0] + cnt_ref[1]
    inv = 1.0 / jnp.maximum(cntv, 1.0)
    mean = a * inv[:, None]
    r = (jnp.dot(mean, wl_ref[...], preferred_element_type=jnp.float32)
         + jnp.dot(x_ref[...], wr_ref[...], preferred_element_type=jnp.float32)
         + b_ref[...])
    o_ref[...] = jnp.maximum(r, 0.0) if relu else r


def _tc_combine(agg2, cnt2, xin, wlT, wrT, b, relu):
    return pl.pallas_call(
        functools.partial(_tc_combine_body, relu=relu),
        grid=(pl.cdiv(_N, _BLK),),
        in_specs=[
            pl.BlockSpec((_NC, _BLK, _D), lambda i: (0, i, 0)),
            pl.BlockSpec((_NC, _BLK), lambda i: (0, i)),
            pl.BlockSpec((_BLK, _D), lambda i: (i, 0)),
            pl.BlockSpec((_D, _D), lambda i: (0, 0)),
            pl.BlockSpec((_D, _D), lambda i: (0, 0)),
            pl.BlockSpec((1, _D), lambda i: (0, 0)),
        ],
        out_specs=pl.BlockSpec((_BLK, _D), lambda i: (i, 0)),
        out_shape=jax.ShapeDtypeStruct((_N, _D), jnp.float32),
    )(agg2, cnt2, xin, wlT, wrT, b)


def kernel(x, edge_index, W1_l, W1_r, b1, W2_l, W2_r, b2):
    zr = jnp.zeros((_CH, _D), jnp.float32)
    zc = jnp.zeros((_RPT,), jnp.float32)

    src = edge_index[0]
    dst = edge_index[1]
    agg1, cnt = _sc_agg_with_counts(x, src, dst, zr, zc)
    h = _tc_combine(agg1.reshape(_NC, _NP, _D), cnt.reshape(_NC, _NP),
                    x, W1_l.T, W1_r.T, b1.reshape(1, _D), relu=True)
    agg2 = _sc_agg(h, src, dst, zr)
    out = _tc_combine(agg2.reshape(_NC, _NP, _D), cnt.reshape(_NC, _NP),
                      h, W2_l.T, W2_r.T, b2.reshape(1, _D), relu=False)
    return out


# speedup vs baseline: 1.2194x; 1.0340x over previous
import functools

import jax
import jax.numpy as jnp
from jax import lax
from jax.experimental import pallas as pl
from jax.experimental.pallas import tpu as pltpu
from jax.experimental.pallas import tpu_sc as plsc

_N = 10000
_NP = 10240
_D = 128
_E = 320000
_NC = 2
_NS = 16
_NW = _NC * _NS
_CH = 64
_K = 160
_RCH = _E // _CH
_RPT = _NP // _NS


_NB = 5


def _sc_agg_body(*refs, with_counts):
    if with_counts:
        (y_hbm, edges_hbm, zr_hbm, zc_hbm, agg_hbm, cnt_hbm,
         acc_sh, cnt_sh, rows_v, sidx, didx, ones_v, gsem, ssem, csem, isem) = refs
    else:
        (y_hbm, edges_hbm, zr_hbm, agg_hbm,
         acc_sh, rows_v, sidx, didx, gsem, ssem, isem) = refs

    c = lax.axis_index("c")
    s = lax.axis_index("s")
    wid = s * _NC + c
    base = s * _RPT
    ebase = wid * _K * _CH
    nchunks = jnp.minimum(_K, jnp.maximum(_RCH - wid * _K, 0))

    pltpu.sync_copy(zr_hbm, rows_v[0])
    for i in range(_RPT // _CH):
        pltpu.sync_copy(rows_v[0], acc_sh.at[pl.ds(base + i * _CH, _CH)])
    if with_counts:
        pltpu.sync_copy(zc_hbm, cnt_sh.at[pl.ds(base, _RPT)])
        for i in range(_CH // 16):
            ones_v[pl.ds(i * 16, 16)] = jnp.full((16,), 1.0, jnp.float32)
    plsc.subcore_barrier()

    def idx_start(j, b, p):
        off = ebase + j * _CH
        pltpu.async_copy(edges_hbm.at[pl.ds(off, _CH)], sidx[p][b], isem[p][b])
        pltpu.async_copy(edges_hbm.at[pl.ds(_E + off, _CH)], didx[p][b], isem[p][b])

    def idx_wait(b, p):
        pltpu.make_async_copy(edges_hbm.at[pl.ds(0, _CH)], sidx[p][b], isem[p][b]).wait()
        pltpu.make_async_copy(edges_hbm.at[pl.ds(0, _CH)], didx[p][b], isem[p][b]).wait()

    def gather_start(b, p):
        pltpu.async_copy(y_hbm.at[sidx[p][b]], rows_v[b], gsem[b])

    def gather_wait(b):
        pltpu.make_async_copy(y_hbm.at[pl.ds(0, _CH)], rows_v[b], gsem[b]).wait()

    def scatter_start(b, p):
        pltpu.async_copy(rows_v[b], acc_sh.at[didx[p][b]], ssem[b], add=True)
        if with_counts:
            pltpu.async_copy(ones_v, cnt_sh.at[didx[p][b]], csem[b], add=True)

    def scatter_wait(b):
        pltpu.make_async_copy(rows_v[b], acc_sh.at[pl.ds(0, _CH)], ssem[b]).wait()
        if with_counts:
            pltpu.make_async_copy(ones_v, cnt_sh.at[pl.ds(0, _CH)], csem[b]).wait()

    def halfgroup(base_chunk, p):
        for b in range(_NB):
            j = base_chunk + b
            gather_wait(b)
            scatter_start(b, p)

            @pl.when(j + _NB < nchunks)
            def _():
                idx_start(j + _NB, b, 1 - p)
        for b in range(_NB):
            j = base_chunk + b
            scatter_wait(b)

            @pl.when(j + _NB < nchunks)
            def _():
                idx_wait(b, 1 - p)
                gather_start(b, 1 - p)

    def group(jj, carry):
        q0 = jj * 2 * _NB
        halfgroup(q0, 0)
        halfgroup(q0 + _NB, 1)
        return carry

    for b in range(_NB):
        idx_start(b, b, 0)
    for b in range(_NB):
        idx_wait(b, 0)
        gather_start(b, 0)
    lax.fori_loop(0, nchunks // (2 * _NB), group, 0)
    plsc.subcore_barrier()

    off = c * _NP + base
    pltpu.sync_copy(acc_sh.at[pl.ds(base, _RPT)], agg_hbm.at[pl.ds(off, _RPT)])
    if with_counts:
        pltpu.sync_copy(cnt_sh.at[pl.ds(base, _RPT)], cnt_hbm.at[pl.ds(off, _RPT)])


_SC_MESH = plsc.VectorSubcoreMesh(core_axis_name="c", subcore_axis_name="s")


def _sc_agg_with_counts(y, edges, zr, zc):
    return pl.kernel(
        functools.partial(_sc_agg_body, with_counts=True),
        out_type=(jax.ShapeDtypeStruct((_NC * _NP, _D), jnp.float32),
                  jax.ShapeDtypeStruct((_NC * _NP,), jnp.float32)),
        mesh=_SC_MESH,
        scratch_types=[
            pltpu.VMEM_SHARED((_NP, _D), jnp.float32),
            pltpu.VMEM_SHARED((_NP,), jnp.float32),
            [pltpu.VMEM((_CH, _D), jnp.float32) for _ in range(_NB)],
            [[pltpu.VMEM((_CH,), jnp.int32) for _ in range(_NB)] for _ in range(2)],
            [[pltpu.VMEM((_CH,), jnp.int32) for _ in range(_NB)] for _ in range(2)],
            pltpu.VMEM((_CH,), jnp.float32),
            [pltpu.SemaphoreType.DMA for _ in range(_NB)],
            [pltpu.SemaphoreType.DMA for _ in range(_NB)],
            [pltpu.SemaphoreType.DMA for _ in range(_NB)],
            [[pltpu.SemaphoreType.DMA for _ in range(_NB)] for _ in range(2)],
        ],
    )(y, edges, zr, zc)


def _sc_agg(y, edges, zr):
    return pl.kernel(
        functools.partial(_sc_agg_body, with_counts=False),
        out_type=jax.ShapeDtypeStruct((_NC * _NP, _D), jnp.float32),
        mesh=_SC_MESH,
        scratch_types=[
            pltpu.VMEM_SHARED((_NP, _D), jnp.float32),
            [pltpu.VMEM((_CH, _D), jnp.float32) for _ in range(_NB)],
            [[pltpu.VMEM((_CH,), jnp.int32) for _ in range(_NB)] for _ in range(2)],
            [[pltpu.VMEM((_CH,), jnp.int32) for _ in range(_NB)] for _ in range(2)],
            [pltpu.SemaphoreType.DMA for _ in range(_NB)],
            [pltpu.SemaphoreType.DMA for _ in range(_NB)],
            [[pltpu.SemaphoreType.DMA for _ in range(_NB)] for _ in range(2)],
        ],
    )(y, edges, zr)


_BLK = 1024


def _tc_combine_body(agg_ref, cnt_ref, x_ref, wl_ref, wr_ref, b_ref, *out_refs, relu):
    a = agg_ref[0] + agg_ref[1]
    cntv = cnt_ref[0] + cnt_ref[1]
    inv = 1.0 / jnp.maximum(cntv, 1.0)
    mean = a * inv[:, None]
    r = (jnp.dot(mean, wl_ref[...], preferred_element_type=jnp.float32)
         + jnp.dot(x_ref[...], wr_ref[...], preferred_element_type=jnp.float32)
         + b_ref[...])
    out_refs[0][...] = jnp.maximum(r, 0.0) if relu else r


def _tc_combine(agg2, cnt2, xin, wlT, wrT, b, relu):
    return pl.pallas_call(
        functools.partial(_tc_combine_body, relu=relu),
        grid=(pl.cdiv(_N, _BLK),),
        in_specs=[
            pl.BlockSpec((_NC, _BLK, _D), lambda i: (0, i, 0)),
            pl.BlockSpec((_NC, _BLK), lambda i: (0, i)),
            pl.BlockSpec((_BLK, _D), lambda i: (i, 0)),
            pl.BlockSpec((_D, _D), lambda i: (0, 0)),
            pl.BlockSpec((_D, _D), lambda i: (0, 0)),
            pl.BlockSpec((1, _D), lambda i: (0, 0)),
        ],
        out_specs=pl.BlockSpec((_BLK, _D), lambda i: (i, 0)),
        out_shape=jax.ShapeDtypeStruct((_N, _D), jnp.float32),
    )(agg2, cnt2, xin, wlT, wrT, b)


def kernel(x, edge_index, W1_l, W1_r, b1, W2_l, W2_r, b2):
    zr = jnp.zeros((_CH, _D), jnp.float32)
    zc = jnp.zeros((_RPT,), jnp.float32)

    edges = edge_index.reshape(2 * _E)
    agg1, cnt = _sc_agg_with_counts(x, edges, zr, zc)
    h = _tc_combine(agg1.reshape(_NC, _NP, _D), cnt.reshape(_NC, _NP),
                    x, W1_l.T, W1_r.T, b1.reshape(1, _D), relu=True)
    agg2 = _sc_agg(h, edges, zr)
    out = _tc_combine(agg2.reshape(_NC, _NP, _D), cnt.reshape(_NC, _NP),
                      h, W2_l.T, W2_r.T, b2.reshape(1, _D), relu=False)
    return out
